# baseline (speedup 1.0000x reference)
import jax
import jax.numpy as jnp
from jax import lax
from jax.experimental import pallas as pl
from jax.experimental.pallas import tpu as pltpu
from jax.experimental.pallas import tpu_sc as plsc

C = 100000
D = 64
B = 16384
LW = 1.0

NC = 2
NS = 16
NW = NC * NS
FPT = D // NW
CH = 1024
NK = B // CH
SPT = B // NS
HSLICE = 6256
HPAD = NS * HSLICE


def _body(ft_hbm, y_hbm, ct_hbm, out_hbm,
          hist_sh, warr_sh,
          acc_v, row_v, yb0_v, yb1_v, fb0_v, fb1_v, hidx_v, ones_v,
          w_v, obuf_v, sem, sema, sem0, sem1):
    c = lax.axis_index("c")
    s = lax.axis_index("s")
    wid = c * NS + s
    f0 = c * (NS * FPT) + s * FPT
    ybufs, fbufs, sems = (yb0_v, yb1_v), (fb0_v, fb1_v), (sem0, sem1)

    cp_row = pltpu.async_copy(ct_hbm.at[f0, pl.ds(0, C)], row_v, sem)
    cp_hidx = [
        pltpu.async_copy(y_hbm.at[pl.ds(s * SPT + j * 128, 128)],
                         hidx_v.at[j], sema)
        for j in range(8)
    ]

    def zfill(g, _):
        for j in range(8):
            acc_v[pl.ds(g * 128 + j * 16, 16)] = jnp.zeros((16,), jnp.float32)
        return 0
    lax.fori_loop(0, HSLICE // 128 + 1, zfill, 0)
    cp_hz = pltpu.async_copy(
        acc_v.at[pl.ds(0, HSLICE)], hist_sh.at[pl.ds(s * HSLICE, HSLICE)], sem0)
    lax.fori_loop(HSLICE // 128 + 1, B // 128, zfill, 0)
    for j in range(128 // 16):
        ones_v[pl.ds(j * 16, 16)] = jnp.ones((16,), jnp.float32)
    cp_hz.wait()
    plsc.subcore_barrier()

    for cp in cp_hidx:
        cp.wait()
    for j in range(8):
        pltpu.sync_copy(ones_v, hist_sh.at[hidx_v.at[j]], add=True)

    cp_row.wait()
    for fi in range(FPT):
        f = f0 + fi
        if fi > 0:
            pltpu.sync_copy(ct_hbm.at[f, pl.ds(0, C)], row_v)
        pend = {}
        pend[0] = (
            pltpu.async_copy(y_hbm.at[pl.ds(0, CH)], ybufs[0], sems[0]),
            pltpu.async_copy(ft_hbm.at[f, pl.ds(0, CH)], fbufs[0], sems[0]),
        )
        for k in range(NK):
            b = k % 2
            if k + 1 < NK:
                pend[1 - b] = (
                    pltpu.async_copy(
                        y_hbm.at[pl.ds((k + 1) * CH, CH)], ybufs[1 - b],
                        sems[1 - b]),
                    pltpu.async_copy(
                        ft_hbm.at[f, pl.ds((k + 1) * CH, CH)], fbufs[1 - b],
                        sems[1 - b]),
                )
            for cp in pend[b]:
                cp.wait()
            yb, fb = ybufs[b], fbufs[b]

            def gbody(g, _, base=k * CH, yb=yb, fb=fb):
                for j in range(8):
                    off = g * 128 + j * 16
                    idx = yb[pl.ds(off, 16)]
                    v = plsc.load_gather(row_v, [idx])
                    fv = fb[pl.ds(off, 16)]
                    d = v - fv
                    plsc.addupdate(acc_v.at[pl.ds(base + off, 16)], d * d)
                return 0
            lax.fori_loop(0, CH // 128, gbody, 0)
    plsc.subcore_barrier()

    pltpu.sync_copy(y_hbm.at[pl.ds(s * SPT, SPT)], yb0_v)
    pltpu.sync_copy(hist_sh.at[yb0_v], fb0_v)

    def wfill(j, _):
        for t in range(4):
            off = j * 64 + t * 16
            cw = fb0_v[pl.ds(off, 16)]
            w_v[pl.ds(off, 16)] = 1.0 / (cw + 1.0)
        return 0
    lax.fori_loop(0, SPT // 64, wfill, 0)
    pltpu.sync_copy(w_v, warr_sh.at[pl.ds(s * SPT, SPT)])
    plsc.subcore_barrier()

    pend = {0: pltpu.async_copy(warr_sh.at[pl.ds(0, SPT)], fbufs[0], sems[0])}
    accv = jnp.zeros((16,), jnp.float32)
    for k in range(B // SPT):
        b = k % 2
        if k + 1 < B // SPT:
            pend[1 - b] = pltpu.async_copy(
                warr_sh.at[pl.ds((k + 1) * SPT, SPT)], fbufs[1 - b],
                sems[1 - b])
        pend[b].wait()
        fb = fbufs[b]

        def rbody(j, a, base=k * SPT, fb=fb):
            for t in range(4):
                off = j * 64 + t * 16
                a = a + acc_v[pl.ds(base + off, 16)] * fb[pl.ds(off, 16)]
            return a
        accv = lax.fori_loop(0, SPT // 64, rbody, accv)

    obuf_v[...] = accv
    pltpu.sync_copy(obuf_v, out_hbm.at[wid])


_mesh = plsc.VectorSubcoreMesh(core_axis_name="c", subcore_axis_name="s")

_sc_call = pl.kernel(
    _body,
    out_type=jax.ShapeDtypeStruct((NW, 16), jnp.float32),
    mesh=_mesh,
    scratch_types=[
        pltpu.VMEM_SHARED((HPAD,), jnp.float32),
        pltpu.VMEM_SHARED((B,), jnp.float32),
        pltpu.VMEM((B,), jnp.float32),
        pltpu.VMEM((C,), jnp.float32),
        pltpu.VMEM((CH,), jnp.int32),
        pltpu.VMEM((CH,), jnp.int32),
        pltpu.VMEM((CH,), jnp.float32),
        pltpu.VMEM((CH,), jnp.float32),
        pltpu.VMEM((8, 128), jnp.int32),
        pltpu.VMEM((128,), jnp.float32),
        pltpu.VMEM((SPT,), jnp.float32),
        pltpu.VMEM((16,), jnp.float32),
        pltpu.SemaphoreType.DMA,
        pltpu.SemaphoreType.DMA,
        pltpu.SemaphoreType.DMA,
        pltpu.SemaphoreType.DMA,
    ],
    compiler_params=pltpu.CompilerParams(needs_layout_passes=False),
)


@jax.jit
def kernel(feat, y, centers):
    partials = _sc_call(feat.T, y, centers.T)
    return LW * 1.0 / jnp.sum(partials)


# speedup vs baseline: 1.1424x; 1.1424x over previous
import jax
import jax.numpy as jnp
from jax import lax
from jax.experimental import pallas as pl
from jax.experimental.pallas import tpu as pltpu
from jax.experimental.pallas import tpu_sc as plsc

C = 100000
D = 64
B = 16384
LW = 1.0

NC = 2
NS = 16
NW = NC * NS
FPT = D // NW
CH = 1024
NK = B // CH
SPT = B // NS
HSLICE = 6256
HPAD = NS * HSLICE


def _body(ft_hbm, y_hbm, ct_hbm, out_hbm,
          hist_sh, warr_sh,
          acc_v, row_v, yb0_v, yb1_v, fb0_v, fb1_v, hidx_v, ones_v,
          w_v, obuf_v, sem, sema, sem0, sem1):
    c = lax.axis_index("c")
    s = lax.axis_index("s")
    wid = c * NS + s
    f0 = c * (NS * FPT) + s * FPT
    ybufs, fbufs, sems = (yb0_v, yb1_v), (fb0_v, fb1_v), (sem0, sem1)

    cp_row = pltpu.async_copy(ct_hbm.at[f0, pl.ds(0, C)], row_v, sem)
    cp_hidx = [
        pltpu.async_copy(y_hbm.at[pl.ds(s * SPT + j * 128, 128)],
                         hidx_v.at[j], sema)
        for j in range(8)
    ]

    def zfill(g, _):
        for j in range(8):
            acc_v[pl.ds(g * 128 + j * 16, 16)] = jnp.zeros((16,), jnp.float32)
        return 0
    lax.fori_loop(0, HSLICE // 128 + 1, zfill, 0)
    cp_hz = pltpu.async_copy(
        acc_v.at[pl.ds(0, HSLICE)], hist_sh.at[pl.ds(s * HSLICE, HSLICE)], sem0)
    lax.fori_loop(HSLICE // 128 + 1, B // 128, zfill, 0)
    for j in range(128 // 16):
        ones_v[pl.ds(j * 16, 16)] = jnp.ones((16,), jnp.float32)
    cp_hz.wait()
    plsc.subcore_barrier()

    for cp in cp_hidx:
        cp.wait()
    for j in range(8):
        pltpu.sync_copy(ones_v, hist_sh.at[hidx_v.at[j]], add=True)

    cp_row.wait()
    for fi in range(FPT):
        f = f0 + fi
        if fi > 0:
            pltpu.sync_copy(ct_hbm.at[f, pl.ds(0, C)], row_v)
        pend = {}
        pend[0] = (
            pltpu.async_copy(y_hbm.at[pl.ds(0, CH)], ybufs[0], sems[0]),
            pltpu.async_copy(ft_hbm.at[f, pl.ds(0, CH)], fbufs[0], sems[0]),
        )
        for k in range(NK):
            b = k % 2
            if k + 1 < NK:
                pend[1 - b] = (
                    pltpu.async_copy(
                        y_hbm.at[pl.ds((k + 1) * CH, CH)], ybufs[1 - b],
                        sems[1 - b]),
                    pltpu.async_copy(
                        ft_hbm.at[f, pl.ds((k + 1) * CH, CH)], fbufs[1 - b],
                        sems[1 - b]),
                )
            for cp in pend[b]:
                cp.wait()
            yb, fb = ybufs[b], fbufs[b]
            base = k * CH

            @plsc.parallel_loop(0, CH, 16, unroll=8)
            def gbody(i, yb=yb, fb=fb, base=base):
                idx = yb[pl.ds(i, 16)]
                v = plsc.load_gather(row_v, [idx])
                fv = fb[pl.ds(i, 16)]
                d = v - fv
                plsc.addupdate(acc_v.at[pl.ds(base + i, 16)], d * d)
    plsc.subcore_barrier()

    pltpu.sync_copy(y_hbm.at[pl.ds(s * SPT, SPT)], yb0_v)
    pltpu.sync_copy(hist_sh.at[yb0_v], fb0_v)

    def wfill(j, _):
        for t in range(4):
            off = j * 64 + t * 16
            cw = fb0_v[pl.ds(off, 16)]
            w_v[pl.ds(off, 16)] = 1.0 / (cw + 1.0)
        return 0
    lax.fori_loop(0, SPT // 64, wfill, 0)
    pltpu.sync_copy(w_v, warr_sh.at[pl.ds(s * SPT, SPT)])
    plsc.subcore_barrier()

    pend = {0: pltpu.async_copy(warr_sh.at[pl.ds(0, SPT)], fbufs[0], sems[0])}
    accv = jnp.zeros((16,), jnp.float32)
    for k in range(B // SPT):
        b = k % 2
        if k + 1 < B // SPT:
            pend[1 - b] = pltpu.async_copy(
                warr_sh.at[pl.ds((k + 1) * SPT, SPT)], fbufs[1 - b],
                sems[1 - b])
        pend[b].wait()
        fb = fbufs[b]

        def rbody(j, a, base=k * SPT, fb=fb):
            for t in range(4):
                off = j * 64 + t * 16
                a = a + acc_v[pl.ds(base + off, 16)] * fb[pl.ds(off, 16)]
            return a
        accv = lax.fori_loop(0, SPT // 64, rbody, accv)

    obuf_v[...] = accv
    pltpu.sync_copy(obuf_v, out_hbm.at[wid])


_mesh = plsc.VectorSubcoreMesh(core_axis_name="c", subcore_axis_name="s")

_sc_call = pl.kernel(
    _body,
    out_type=jax.ShapeDtypeStruct((NW, 16), jnp.float32),
    mesh=_mesh,
    scratch_types=[
        pltpu.VMEM_SHARED((HPAD,), jnp.float32),
        pltpu.VMEM_SHARED((B,), jnp.float32),
        pltpu.VMEM((B,), jnp.float32),
        pltpu.VMEM((C,), jnp.float32),
        pltpu.VMEM((CH,), jnp.int32),
        pltpu.VMEM((CH,), jnp.int32),
        pltpu.VMEM((CH,), jnp.float32),
        pltpu.VMEM((CH,), jnp.float32),
        pltpu.VMEM((8, 128), jnp.int32),
        pltpu.VMEM((128,), jnp.float32),
        pltpu.VMEM((SPT,), jnp.float32),
        pltpu.VMEM((16,), jnp.float32),
        pltpu.SemaphoreType.DMA,
        pltpu.SemaphoreType.DMA,
        pltpu.SemaphoreType.DMA,
        pltpu.SemaphoreType.DMA,
    ],
    compiler_params=pltpu.CompilerParams(needs_layout_passes=False),
)


@jax.jit
def kernel(feat, y, centers):
    partials = _sc_call(feat.T, y, centers.T)
    return LW * 1.0 / jnp.sum(partials)


# speedup vs baseline: 1.1511x; 1.0077x over previous
import jax
import jax.numpy as jnp
from jax import lax
from jax.experimental import pallas as pl
from jax.experimental.pallas import tpu as pltpu
from jax.experimental.pallas import tpu_sc as plsc

C = 100000
D = 64
B = 16384
LW = 1.0

NC = 2
NS = 16
NW = NC * NS
FPT = D // NW
CH = 1024
NK = B // CH
SPT = B // NS
HSLICE = 6256
HPAD = NS * HSLICE


def _body(ft_hbm, y_hbm, ct_hbm, out_hbm,
          hist_sh, warr_sh,
          acc_v, row_v, yb0_v, yb1_v, fb0_v, fb1_v, ones_v,
          w_v, obuf_v, sem, sema, sem0, sem1):
    c = lax.axis_index("c")
    s = lax.axis_index("s")
    wid = c * NS + s
    f0 = c * (NS * FPT) + s * FPT
    ybufs, fbufs, sems = (yb0_v, yb1_v), (fb0_v, fb1_v), (sem0, sem1)

    cp_row = pltpu.async_copy(ct_hbm.at[f0, pl.ds(0, C)], row_v, sem)
    cp_hidx = pltpu.async_copy(y_hbm.at[pl.ds(s * SPT, SPT)], yb0_v, sema)

    @plsc.parallel_loop(0, HSLICE + 128, 16, unroll=8)
    def zfill0(i):
        acc_v[pl.ds(i, 16)] = jnp.zeros((16,), jnp.float32)
    cp_hz = pltpu.async_copy(
        acc_v.at[pl.ds(0, HSLICE)], hist_sh.at[pl.ds(s * HSLICE, HSLICE)], sem0)

    @plsc.parallel_loop(HSLICE + 128, B, 16, unroll=8)
    def zfill1(i):
        acc_v[pl.ds(i, 16)] = jnp.zeros((16,), jnp.float32)

    @plsc.parallel_loop(0, SPT, 16, unroll=8)
    def ofill(i):
        ones_v[pl.ds(i, 16)] = jnp.ones((16,), jnp.float32)
    cp_hz.wait()
    plsc.subcore_barrier()

    cp_hidx.wait()
    pltpu.sync_copy(ones_v, hist_sh.at[yb0_v], add=True)

    cp_row.wait()
    for fi in range(FPT):
        f = f0 + fi
        if fi > 0:
            pltpu.sync_copy(ct_hbm.at[f, pl.ds(0, C)], row_v)
        pend = {}
        pend[0] = (
            pltpu.async_copy(y_hbm.at[pl.ds(0, CH)], ybufs[0], sems[0]),
            pltpu.async_copy(ft_hbm.at[f, pl.ds(0, CH)], fbufs[0], sems[0]),
        )
        for k in range(NK):
            b = k % 2
            if k + 1 < NK:
                pend[1 - b] = (
                    pltpu.async_copy(
                        y_hbm.at[pl.ds((k + 1) * CH, CH)], ybufs[1 - b],
                        sems[1 - b]),
                    pltpu.async_copy(
                        ft_hbm.at[f, pl.ds((k + 1) * CH, CH)], fbufs[1 - b],
                        sems[1 - b]),
                )
            for cp in pend[b]:
                cp.wait()
            yb, fb = ybufs[b], fbufs[b]
            base = k * CH

            @plsc.parallel_loop(0, CH, 16, unroll=8)
            def gbody(i, yb=yb, fb=fb, base=base):
                idx = yb[pl.ds(i, 16)]
                v = plsc.load_gather(row_v, [idx])
                fv = fb[pl.ds(i, 16)]
                d = v - fv
                plsc.addupdate(acc_v.at[pl.ds(base + i, 16)], d * d)
    plsc.subcore_barrier()

    pltpu.sync_copy(y_hbm.at[pl.ds(s * SPT, SPT)], yb0_v)
    pltpu.sync_copy(hist_sh.at[yb0_v], fb0_v)

    def wfill(j, _):
        for t in range(4):
            off = j * 64 + t * 16
            cw = fb0_v[pl.ds(off, 16)]
            w_v[pl.ds(off, 16)] = 1.0 / (cw + 1.0)
        return 0
    lax.fori_loop(0, SPT // 64, wfill, 0)
    pltpu.sync_copy(w_v, warr_sh.at[pl.ds(s * SPT, SPT)])
    plsc.subcore_barrier()

    pend = {0: pltpu.async_copy(warr_sh.at[pl.ds(0, SPT)], fbufs[0], sems[0])}
    accv = jnp.zeros((16,), jnp.float32)
    for k in range(B // SPT):
        b = k % 2
        if k + 1 < B // SPT:
            pend[1 - b] = pltpu.async_copy(
                warr_sh.at[pl.ds((k + 1) * SPT, SPT)], fbufs[1 - b],
                sems[1 - b])
        pend[b].wait()
        fb = fbufs[b]

        def rbody(j, a, base=k * SPT, fb=fb):
            for t in range(4):
                off = j * 64 + t * 16
                a = a + acc_v[pl.ds(base + off, 16)] * fb[pl.ds(off, 16)]
            return a
        accv = lax.fori_loop(0, SPT // 64, rbody, accv)

    obuf_v[...] = accv
    pltpu.sync_copy(obuf_v, out_hbm.at[wid])


_mesh = plsc.VectorSubcoreMesh(core_axis_name="c", subcore_axis_name="s")

_sc_call = pl.kernel(
    _body,
    out_type=jax.ShapeDtypeStruct((NW, 16), jnp.float32),
    mesh=_mesh,
    scratch_types=[
        pltpu.VMEM_SHARED((HPAD,), jnp.float32),
        pltpu.VMEM_SHARED((B,), jnp.float32),
        pltpu.VMEM((B,), jnp.float32),
        pltpu.VMEM((C,), jnp.float32),
        pltpu.VMEM((CH,), jnp.int32),
        pltpu.VMEM((CH,), jnp.int32),
        pltpu.VMEM((CH,), jnp.float32),
        pltpu.VMEM((CH,), jnp.float32),
        pltpu.VMEM((SPT,), jnp.float32),
        pltpu.VMEM((SPT,), jnp.float32),
        pltpu.VMEM((16,), jnp.float32),
        pltpu.SemaphoreType.DMA,
        pltpu.SemaphoreType.DMA,
        pltpu.SemaphoreType.DMA,
        pltpu.SemaphoreType.DMA,
    ],
    compiler_params=pltpu.CompilerParams(needs_layout_passes=False),
)


@jax.jit
def kernel(feat, y, centers):
    partials = _sc_call(feat.T, y, centers.T)
    return LW * 1.0 / jnp.sum(partials)


# speedup vs baseline: 1.1997x; 1.0422x over previous
import jax
import jax.numpy as jnp
from jax import lax
from jax.experimental import pallas as pl
from jax.experimental.pallas import tpu as pltpu
from jax.experimental.pallas import tpu_sc as plsc

C = 100000
D = 64
B = 16384
LW = 1.0

NC = 2
NS = 16
NW = NC * NS
FPT = D // NW
CH = 1024
NK = B // CH
SPT = B // NS
HSLICE = 6256
HPAD = NS * HSLICE


def _body(ft_hbm, y_hbm, ct_hbm, out_hbm,
          hist_sh, warr_sh,
          row_v, zb_v, ones_v, yb0_v, yb1_v, fb0_v, fb1_v, wb0_v, wb1_v,
          w_v, obuf_v, sem, sema, sem0, sem1):
    c = lax.axis_index("c")
    s = lax.axis_index("s")
    wid = c * NS + s
    f0 = c * (NS * FPT) + s * FPT
    ybufs, fbufs, wbufs = (yb0_v, yb1_v), (fb0_v, fb1_v), (wb0_v, wb1_v)
    sems = (sem0, sem1)

    cp_row = pltpu.async_copy(ct_hbm.at[f0, pl.ds(0, C)], row_v, sem)
    cp_hidx = pltpu.async_copy(y_hbm.at[pl.ds(s * SPT, SPT)], yb0_v, sema)

    @plsc.parallel_loop(0, HSLICE, 16, unroll=8)
    def zfill(i):
        zb_v[pl.ds(i, 16)] = jnp.zeros((16,), jnp.float32)

    @plsc.parallel_loop(0, SPT, 16, unroll=8)
    def ofill(i):
        ones_v[pl.ds(i, 16)] = jnp.ones((16,), jnp.float32)

    pltpu.sync_copy(zb_v, hist_sh.at[pl.ds(s * HSLICE, HSLICE)])
    plsc.subcore_barrier()

    cp_hidx.wait()
    pltpu.sync_copy(ones_v, hist_sh.at[yb0_v], add=True)
    cp_row.wait()
    plsc.subcore_barrier()

    pltpu.sync_copy(hist_sh.at[yb0_v], fb0_v)

    def wfill(j, _):
        for t in range(4):
            off = j * 64 + t * 16
            cw = fb0_v[pl.ds(off, 16)]
            w_v[pl.ds(off, 16)] = 1.0 / (cw + 1.0)
        return 0
    lax.fori_loop(0, SPT // 64, wfill, 0)
    pltpu.sync_copy(w_v, warr_sh.at[pl.ds(s * SPT, SPT)])
    plsc.subcore_barrier()

    acc = jnp.zeros((16,), jnp.float32)
    for fi in range(FPT):
        f = f0 + fi
        if fi > 0:
            pltpu.sync_copy(ct_hbm.at[f, pl.ds(0, C)], row_v)
        pend = {}
        pend[0] = (
            pltpu.async_copy(y_hbm.at[pl.ds(0, CH)], ybufs[0], sems[0]),
            pltpu.async_copy(ft_hbm.at[f, pl.ds(0, CH)], fbufs[0], sems[0]),
        )
        for k in range(NK):
            b = k % 2
            if k + 1 < NK:
                pend[1 - b] = (
                    pltpu.async_copy(
                        y_hbm.at[pl.ds((k + 1) * CH, CH)], ybufs[1 - b],
                        sems[1 - b]),
                    pltpu.async_copy(
                        ft_hbm.at[f, pl.ds((k + 1) * CH, CH)], fbufs[1 - b],
                        sems[1 - b]),
                )
            pltpu.sync_copy(warr_sh.at[pl.ds(k * CH, CH)], wbufs[b])
            for cp in pend[b]:
                cp.wait()
            yb, fb, wb = ybufs[b], fbufs[b], wbufs[b]

            @plsc.parallel_loop(0, CH, 16, unroll=8, carry=acc)
            def gbody(i, a, yb=yb, fb=fb, wb=wb):
                idx = yb[pl.ds(i, 16)]
                v = plsc.load_gather(row_v, [idx])
                fv = fb[pl.ds(i, 16)]
                wv = wb[pl.ds(i, 16)]
                d = v - fv
                return a + d * d * wv
            acc = gbody
    obuf_v[...] = acc
    pltpu.sync_copy(obuf_v, out_hbm.at[wid])


_mesh = plsc.VectorSubcoreMesh(core_axis_name="c", subcore_axis_name="s")

_sc_call = pl.kernel(
    _body,
    out_type=jax.ShapeDtypeStruct((NW, 16), jnp.float32),
    mesh=_mesh,
    scratch_types=[
        pltpu.VMEM_SHARED((HPAD,), jnp.float32),
        pltpu.VMEM_SHARED((B,), jnp.float32),
        pltpu.VMEM((C,), jnp.float32),
        pltpu.VMEM((HSLICE,), jnp.float32),
        pltpu.VMEM((SPT,), jnp.float32),
        pltpu.VMEM((CH,), jnp.int32),
        pltpu.VMEM((CH,), jnp.int32),
        pltpu.VMEM((CH,), jnp.float32),
        pltpu.VMEM((CH,), jnp.float32),
        pltpu.VMEM((CH,), jnp.float32),
        pltpu.VMEM((CH,), jnp.float32),
        pltpu.VMEM((SPT,), jnp.float32),
        pltpu.VMEM((16,), jnp.float32),
        pltpu.SemaphoreType.DMA,
        pltpu.SemaphoreType.DMA,
        pltpu.SemaphoreType.DMA,
        pltpu.SemaphoreType.DMA,
    ],
    compiler_params=pltpu.CompilerParams(needs_layout_passes=False),
)


@jax.jit
def kernel(feat, y, centers):
    partials = _sc_call(feat.T, y, centers.T)
    return LW * 1.0 / jnp.sum(partials)


# speedup vs baseline: 1.3763x; 1.1472x over previous
import jax
import jax.numpy as jnp
from jax import lax
from jax.experimental import pallas as pl
from jax.experimental.pallas import tpu as pltpu
from jax.experimental.pallas import tpu_sc as plsc

C = 100000
D = 64
B = 16384
LW = 1.0

NC = 2
NS = 16
NW = NC * NS
FPT = D // NW
CH = 2048
NK = B // CH
SPT = B // NS
HSLICE = 6256
HPAD = NS * HSLICE


def _body(ft_hbm, y_hbm, ct_hbm, out_hbm,
          hist_sh, warr_sh,
          row_v, zb_v, ones_v, sidx_v, yb0_v, yb1_v, fb0_v, fb1_v,
          wb0_v, wb1_v, w_v, obuf_v, sem, sema, sem0, sem1):
    c = lax.axis_index("c")
    s = lax.axis_index("s")
    wid = c * NS + s
    f0 = c * (NS * FPT) + s * FPT
    ybufs, fbufs, wbufs = (yb0_v, yb1_v), (fb0_v, fb1_v), (wb0_v, wb1_v)
    sems = (sem0, sem1)

    cp_row = pltpu.async_copy(ct_hbm.at[f0, pl.ds(0, C)], row_v, sem)
    cp_hidx = pltpu.async_copy(y_hbm.at[pl.ds(s * SPT, SPT)], sidx_v, sema)

    @plsc.parallel_loop(0, HSLICE, 16, unroll=8)
    def zfill(i):
        zb_v[pl.ds(i, 16)] = jnp.zeros((16,), jnp.float32)

    @plsc.parallel_loop(0, SPT, 16, unroll=8)
    def ofill(i):
        ones_v[pl.ds(i, 16)] = jnp.ones((16,), jnp.float32)

    pltpu.sync_copy(zb_v, hist_sh.at[pl.ds(s * HSLICE, HSLICE)])
    plsc.subcore_barrier()

    cp_hidx.wait()
    pltpu.sync_copy(ones_v, hist_sh.at[sidx_v], add=True)
    cp_row.wait()
    plsc.subcore_barrier()

    pltpu.sync_copy(hist_sh.at[sidx_v], fb0_v.at[pl.ds(0, SPT)])

    def wfill(j, _):
        for t in range(4):
            off = j * 64 + t * 16
            cw = fb0_v[pl.ds(off, 16)]
            w_v[pl.ds(off, 16)] = 1.0 / (cw + 1.0)
        return 0
    lax.fori_loop(0, SPT // 64, wfill, 0)
    pltpu.sync_copy(w_v, warr_sh.at[pl.ds(s * SPT, SPT)])
    plsc.subcore_barrier()

    acc = jnp.zeros((16,), jnp.float32)
    for fi in range(FPT):
        f = f0 + fi
        if fi > 0:
            pltpu.sync_copy(ct_hbm.at[f, pl.ds(0, C)], row_v)
        pend = {}
        pend[0] = (
            pltpu.async_copy(y_hbm.at[pl.ds(0, CH)], ybufs[0], sems[0]),
            pltpu.async_copy(ft_hbm.at[f, pl.ds(0, CH)], fbufs[0], sems[0]),
        )
        for k in range(NK):
            b = k % 2
            if k + 1 < NK:
                pend[1 - b] = (
                    pltpu.async_copy(
                        y_hbm.at[pl.ds((k + 1) * CH, CH)], ybufs[1 - b],
                        sems[1 - b]),
                    pltpu.async_copy(
                        ft_hbm.at[f, pl.ds((k + 1) * CH, CH)], fbufs[1 - b],
                        sems[1 - b]),
                )
            pltpu.sync_copy(warr_sh.at[pl.ds(k * CH, CH)], wbufs[b])
            for cp in pend[b]:
                cp.wait()
            yb, fb, wb = ybufs[b], fbufs[b], wbufs[b]

            @plsc.parallel_loop(0, CH, 16, unroll=8, carry=acc)
            def gbody(i, a, yb=yb, fb=fb, wb=wb):
                idx = yb[pl.ds(i, 16)]
                v = plsc.load_gather(row_v, [idx])
                fv = fb[pl.ds(i, 16)]
                wv = wb[pl.ds(i, 16)]
                d = v - fv
                return a + d * d * wv
            acc = gbody
    obuf_v[...] = acc
    pltpu.sync_copy(obuf_v, out_hbm.at[wid])


_mesh = plsc.VectorSubcoreMesh(core_axis_name="c", subcore_axis_name="s")

_sc_call = pl.kernel(
    _body,
    out_type=jax.ShapeDtypeStruct((NW, 16), jnp.float32),
    mesh=_mesh,
    scratch_types=[
        pltpu.VMEM_SHARED((HPAD,), jnp.float32),
        pltpu.VMEM_SHARED((B,), jnp.float32),
        pltpu.VMEM((C,), jnp.float32),
        pltpu.VMEM((HSLICE,), jnp.float32),
        pltpu.VMEM((SPT,), jnp.float32),
        pltpu.VMEM((SPT,), jnp.int32),
        pltpu.VMEM((CH,), jnp.int32),
        pltpu.VMEM((CH,), jnp.int32),
        pltpu.VMEM((CH,), jnp.float32),
        pltpu.VMEM((CH,), jnp.float32),
        pltpu.VMEM((CH,), jnp.float32),
        pltpu.VMEM((CH,), jnp.float32),
        pltpu.VMEM((SPT,), jnp.float32),
        pltpu.VMEM((16,), jnp.float32),
        pltpu.SemaphoreType.DMA,
        pltpu.SemaphoreType.DMA,
        pltpu.SemaphoreType.DMA,
        pltpu.SemaphoreType.DMA,
    ],
    compiler_params=pltpu.CompilerParams(needs_layout_passes=False),
)


@jax.jit
def kernel(feat, y, centers):
    partials = _sc_call(feat.T, y, centers.T)
    return LW * 1.0 / jnp.sum(partials)
